# Initial kernel scaffold; baseline (speedup 1.0000x reference)
import jax
import jax.numpy as jnp
from jax.experimental import pallas as pl


def kernel(x, edge_index, table, W_gcn, b_gcn, W_lin, b_lin):
    raise NotImplementedError("write your pallas kernel here")


# speedup vs baseline: 14.7772x; 14.7772x over previous
import functools

import jax
import jax.numpy as jnp
from jax import lax
from jax.experimental import pallas as pl
from jax.experimental.pallas import tpu as pltpu
from jax.experimental.pallas import tpu_sc as plsc

N = 50000
E = 800000
VOCAB = 100000
D_EMB = 64
D_HID = 128
D_OUT = 64

NC = 2
NS = 16
NW = NC * NS

NPAD = 53248
PT_N = NPAD // NW
HALF = NPAD // NC
ACC_ROWS = HALF + 16
STRIPE = ACC_ROWS // NS

EPAD = 802816
PT_E_DEG = EPAD // NW
PT_E = EPAD // NS

_MESH = dict(core_axis_name="c", subcore_axis_name="s", num_cores=NC,
             num_subcores=NS)


DEG_ROWS = NPAD + 128
DEG_STRIPE = DEG_ROWS // NS


def _sc_gather_deg(xp_hbm, table_hbm, dst_hbm, h_out, deg_out,
                   idx_v, rows_v, dstb_v, idxb_v, ones_v, zb_v, deg_sh, sem):
  c = lax.axis_index("c")
  s = lax.axis_index("s")
  wid = s * NC + c

  zeros16 = jnp.zeros((16,), jnp.float32)
  ones16 = jnp.ones((16,), jnp.float32)
  def _zero(i, _):
    zb_v[pl.ds(i * 16, 16)] = zeros16
    return 0
  lax.fori_loop(0, 1024 // 16, _zero, 0)
  for j in range(8):
    ones_v[pl.ds(j * 16, 16)] = ones16
  for off, sz in ((0, 1024), (1024, 1024), (2048, 1024), (3072, 264)):
    pltpu.sync_copy(zb_v.at[pl.ds(0, sz)],
                    deg_sh.at[pl.ds(s * DEG_STRIPE + off, sz)])
  plsc.subcore_barrier()

  base_n = wid * PT_N
  for k in range(PT_N // 128):
    off = base_n + k * 128
    pltpu.sync_copy(xp_hbm.at[pl.ds(off, 128)], idx_v)
    pltpu.async_copy(table_hbm.at[idx_v], rows_v, sem).wait()
    pltpu.sync_copy(rows_v, h_out.at[pl.ds(off, 128)])

  base_e = (s * NC + c) * PT_E_DEG
  def _hist(k, _):
    pltpu.sync_copy(dst_hbm.at[pl.ds(base_e + k * 128, 128)], dstb_v)
    for j in range(8):
      d = dstb_v[pl.ds(j * 16, 16)]
      idxb_v[pl.ds(j * 16, 16)] = jnp.where(d >= 0, d, NPAD)
    pltpu.sync_copy(ones_v, deg_sh.at[idxb_v], add=True)
    return 0
  lax.fori_loop(0, PT_E_DEG // 128, _hist, 0)
  plsc.subcore_barrier()

  dbase = s * (NPAD // NS)
  for off in (0, 1024, 2048):
    sz = 1024 if off < 2048 else NPAD // NS - 2048
    pltpu.sync_copy(deg_sh.at[pl.ds(dbase + off, sz)], zb_v.at[pl.ds(0, sz)])
    pltpu.sync_copy(zb_v.at[pl.ds(0, sz)],
                    deg_out.at[c].at[pl.ds(dbase + off, sz)])


_gather_deg = pl.kernel(
    _sc_gather_deg,
    out_type=(jax.ShapeDtypeStruct((NPAD, D_EMB), jnp.float32),
              jax.ShapeDtypeStruct((NC, NPAD), jnp.float32)),
    mesh=plsc.VectorSubcoreMesh(**_MESH),
    scratch_types=[
        pltpu.VMEM((128,), jnp.int32),
        pltpu.VMEM((128, D_EMB), jnp.float32),
        pltpu.VMEM((128,), jnp.int32),
        pltpu.VMEM((128,), jnp.int32),
        pltpu.VMEM((128,), jnp.float32),
        pltpu.VMEM((1024,), jnp.float32),
        pltpu.VMEM_SHARED((DEG_ROWS,), jnp.float32),
        pltpu.SemaphoreType.DMA,
    ],
    compiler_params=pltpu.CompilerParams(use_tc_tiling_on_sc=False),
)


_BK = 4096
_GB = NPAD // _BK


def _tc_prep(degT_ref, h_ref, dinv_ref, ht_ref):
  deg = jnp.sum(degT_ref[...], axis=1, keepdims=True) + 1.0
  dinv = lax.rsqrt(deg)
  dinv_ref[...] = dinv
  ht_ref[...] = h_ref[...] * dinv


_prep = pl.pallas_call(
    _tc_prep,
    grid=(_GB,),
    in_specs=[
        pl.BlockSpec((_BK, NC), lambda g: (g, 0)),
        pl.BlockSpec((_BK, D_EMB), lambda g: (g, 0)),
    ],
    out_specs=[
        pl.BlockSpec((_BK, 1), lambda g: (g, 0)),
        pl.BlockSpec((_BK, D_EMB), lambda g: (g, 0)),
    ],
    out_shape=[
        jax.ShapeDtypeStruct((NPAD, 1), jnp.float32),
        jax.ShapeDtypeStruct((NPAD, D_EMB), jnp.float32),
    ],
)


def _sc_edges(ht_hbm, src_hbm, dst_hbm, acc_out,
              srcb, dstb, idxb, rows_v, acc_sh, sem):
  c = lax.axis_index("c")
  s = lax.axis_index("s")
  base_node = c * HALF

  zeros16 = jnp.zeros((16,), jnp.float32)
  def _zero(i, _):
    r = i // 4
    col = i % 4
    rows_v[r, pl.ds(col * 16, 16)] = zeros16
    return 0
  lax.fori_loop(0, 128 * 4, _zero, 0)
  def _zcopy(k, _):
    pltpu.sync_copy(rows_v.at[pl.ds(0, 128)],
                    acc_sh.at[pl.ds(s * STRIPE + k * 128, 128)])
    return 0
  lax.fori_loop(0, 13, _zcopy, 0)
  pltpu.sync_copy(rows_v.at[pl.ds(0, 1)],
                  acc_sh.at[pl.ds(s * STRIPE + 1664, 1)])
  plsc.subcore_barrier()

  base_e = s * PT_E
  def _edge(k, _):
    e0 = base_e + k * 128
    pltpu.sync_copy(src_hbm.at[pl.ds(e0, 128)], srcb)
    pltpu.sync_copy(dst_hbm.at[pl.ds(e0, 128)], dstb)
    gat = pltpu.async_copy(ht_hbm.at[srcb], rows_v, sem)
    for j in range(8):
      d = dstb[pl.ds(j * 16, 16)]
      loc = d - base_node
      ok = (loc >= 0) & (loc < HALF)
      idxb[pl.ds(j * 16, 16)] = jnp.where(ok, loc, HALF)
    gat.wait()
    pltpu.sync_copy(rows_v, acc_sh.at[idxb], add=True)
    return 0
  lax.fori_loop(0, PT_E // 128, _edge, 0)
  plsc.subcore_barrier()

  out_base = c * HALF + s * (HALF // NS)
  sp_base = s * (HALF // NS)
  def _wb(k, _):
    pltpu.sync_copy(acc_sh.at[pl.ds(sp_base + k * 128, 128)],
                    rows_v.at[pl.ds(0, 128)])
    pltpu.sync_copy(rows_v.at[pl.ds(0, 128)],
                    acc_out.at[pl.ds(out_base + k * 128, 128)])
    return 0
  lax.fori_loop(0, 13, _wb, 0)


_edges = pl.kernel(
    _sc_edges,
    out_type=jax.ShapeDtypeStruct((NPAD, D_EMB), jnp.float32),
    mesh=plsc.VectorSubcoreMesh(**_MESH),
    scratch_types=[
        pltpu.VMEM((128,), jnp.int32),
        pltpu.VMEM((128,), jnp.int32),
        pltpu.VMEM((128,), jnp.int32),
        pltpu.VMEM((128, D_EMB), jnp.float32),
        pltpu.VMEM_SHARED((ACC_ROWS, D_EMB), jnp.float32),
        pltpu.SemaphoreType.DMA,
    ],
    compiler_params=pltpu.CompilerParams(use_tc_tiling_on_sc=False),
)


def _tc_final(acc_ref, ht_ref, dinv_ref, wg_ref, bg_ref, wl_ref, bl_ref,
              out_ref, s_scr):
  g = pl.program_id(0)

  @pl.when(g == 0)
  def _():
    s_scr[...] = jnp.zeros_like(s_scr)

  agg = (acc_ref[...] + ht_ref[...]) * dinv_ref[...]
  aggh = jnp.dot(agg, wg_ref[...], preferred_element_type=jnp.float32)
  r = jnp.maximum(aggh + bg_ref[...], 0.0)
  rows = lax.broadcasted_iota(jnp.int32, (_BK, 1), 0) + g * _BK
  r = jnp.where(rows < N, r, 0.0)
  s_scr[...] += jnp.sum(r, axis=0, keepdims=True)

  @pl.when(g == _GB - 1)
  def _():
    out_ref[...] = (
        jnp.dot(s_scr[...], wl_ref[...], preferred_element_type=jnp.float32)
        + jnp.float32(N) * bl_ref[...])


_final = pl.pallas_call(
    _tc_final,
    grid=(_GB,),
    in_specs=[
        pl.BlockSpec((_BK, D_EMB), lambda g: (g, 0)),
        pl.BlockSpec((_BK, D_EMB), lambda g: (g, 0)),
        pl.BlockSpec((_BK, 1), lambda g: (g, 0)),
        pl.BlockSpec((D_EMB, D_HID), lambda g: (0, 0)),
        pl.BlockSpec((1, D_HID), lambda g: (0, 0)),
        pl.BlockSpec((D_HID, D_OUT), lambda g: (0, 0)),
        pl.BlockSpec((1, D_OUT), lambda g: (0, 0)),
    ],
    out_specs=pl.BlockSpec((1, D_OUT), lambda g: (0, 0)),
    out_shape=jax.ShapeDtypeStruct((1, D_OUT), jnp.float32),
    scratch_shapes=[pltpu.VMEM((1, D_HID), jnp.float32)],
)


@jax.jit
def kernel(x, edge_index, table, W_gcn, b_gcn, W_lin, b_lin):
  xp = jnp.concatenate([x[:, 0], jnp.zeros((NPAD - N,), jnp.int32)])
  src = jnp.concatenate([edge_index[0], jnp.zeros((EPAD - E,), jnp.int32)])
  dst = jnp.concatenate(
      [edge_index[1], jnp.full((EPAD - E,), -1, jnp.int32)])

  h, deg_part = _gather_deg(xp, table, dst)
  dinv, ht = _prep(deg_part.T, h)
  acc = _edges(ht, src, dst)
  out = _final(acc, ht, dinv, W_gcn, b_gcn.reshape(1, D_HID),
               W_lin, b_lin.reshape(1, D_OUT))
  return out.reshape(1, 1, D_OUT)


# speedup vs baseline: 19.1084x; 1.2931x over previous
import functools

import jax
import jax.numpy as jnp
from jax import lax
from jax.experimental import pallas as pl
from jax.experimental.pallas import tpu as pltpu
from jax.experimental.pallas import tpu_sc as plsc

N = 50000
E = 800000
VOCAB = 100000
D_EMB = 64
D_HID = 128
D_OUT = 64

NC = 2
NS = 16
NW = NC * NS

NPAD = 53248
PT_N = NPAD // NW
HALF = NPAD // NC
ACC_ROWS = HALF + 16
STRIPE = ACC_ROWS // NS

EPAD = 802816
PT_E_DEG = EPAD // NW
PT_E = EPAD // NS

_MESH = dict(core_axis_name="c", subcore_axis_name="s", num_cores=NC,
             num_subcores=NS)


DEG_ROWS = NPAD + 128
DEG_STRIPE = DEG_ROWS // NS


def _sc_gather_deg(xp_hbm, table_hbm, dst_hbm, h_out, deg_out,
                   idx_v, rows_v, dstb_v, idxb_v, ones_v, zb_v, deg_sh, sem):
  c = lax.axis_index("c")
  s = lax.axis_index("s")
  wid = s * NC + c

  zeros16 = jnp.zeros((16,), jnp.float32)
  ones16 = jnp.ones((16,), jnp.float32)
  def _zero(i, _):
    zb_v[pl.ds(i * 16, 16)] = zeros16
    return 0
  lax.fori_loop(0, 1024 // 16, _zero, 0)
  for j in range(8):
    ones_v[pl.ds(j * 16, 16)] = ones16
  for off, sz in ((0, 1024), (1024, 1024), (2048, 1024), (3072, 264)):
    pltpu.sync_copy(zb_v.at[pl.ds(0, sz)],
                    deg_sh.at[pl.ds(s * DEG_STRIPE + off, sz)])
  plsc.subcore_barrier()

  base_n = wid * PT_N
  for k in range(PT_N // 128):
    off = base_n + k * 128
    pltpu.sync_copy(xp_hbm.at[pl.ds(off, 128)], idx_v)
    pltpu.async_copy(table_hbm.at[idx_v], rows_v, sem).wait()
    pltpu.sync_copy(rows_v, h_out.at[pl.ds(off, 128)])

  base_e = (s * NC + c) * PT_E_DEG
  def _hist(k, _):
    pltpu.sync_copy(dst_hbm.at[pl.ds(base_e + k * 128, 128)], dstb_v)
    for j in range(8):
      d = dstb_v[pl.ds(j * 16, 16)]
      idxb_v[pl.ds(j * 16, 16)] = jnp.where(d >= 0, d, NPAD)
    pltpu.sync_copy(ones_v, deg_sh.at[idxb_v], add=True)
    return 0
  lax.fori_loop(0, PT_E_DEG // 128, _hist, 0)
  plsc.subcore_barrier()

  dbase = s * (NPAD // NS)
  for off in (0, 1024, 2048):
    sz = 1024 if off < 2048 else NPAD // NS - 2048
    pltpu.sync_copy(deg_sh.at[pl.ds(dbase + off, sz)], zb_v.at[pl.ds(0, sz)])
    pltpu.sync_copy(zb_v.at[pl.ds(0, sz)],
                    deg_out.at[c].at[pl.ds(dbase + off, sz)])


_gather_deg = pl.kernel(
    _sc_gather_deg,
    out_type=(jax.ShapeDtypeStruct((NPAD, D_EMB), jnp.float32),
              jax.ShapeDtypeStruct((NC, NPAD), jnp.float32)),
    mesh=plsc.VectorSubcoreMesh(**_MESH),
    scratch_types=[
        pltpu.VMEM((128,), jnp.int32),
        pltpu.VMEM((128, D_EMB), jnp.float32),
        pltpu.VMEM((128,), jnp.int32),
        pltpu.VMEM((128,), jnp.int32),
        pltpu.VMEM((128,), jnp.float32),
        pltpu.VMEM((1024,), jnp.float32),
        pltpu.VMEM_SHARED((DEG_ROWS,), jnp.float32),
        pltpu.SemaphoreType.DMA,
    ],
    compiler_params=pltpu.CompilerParams(use_tc_tiling_on_sc=False),
)


_BK = 4096
_GB = NPAD // _BK


def _tc_prep(degT_ref, h_ref, dinv_ref, ht_ref):
  deg = jnp.sum(degT_ref[...], axis=1, keepdims=True) + 1.0
  dinv = lax.rsqrt(deg)
  dinv_ref[...] = dinv
  ht_ref[...] = h_ref[...] * dinv


_prep = pl.pallas_call(
    _tc_prep,
    grid=(_GB,),
    in_specs=[
        pl.BlockSpec((_BK, NC), lambda g: (g, 0)),
        pl.BlockSpec((_BK, D_EMB), lambda g: (g, 0)),
    ],
    out_specs=[
        pl.BlockSpec((_BK, 1), lambda g: (g, 0)),
        pl.BlockSpec((_BK, D_EMB), lambda g: (g, 0)),
    ],
    out_shape=[
        jax.ShapeDtypeStruct((NPAD, 1), jnp.float32),
        jax.ShapeDtypeStruct((NPAD, D_EMB), jnp.float32),
    ],
)


def _sc_edges(ht_hbm, src_hbm, dst_hbm, acc_out,
              srcb, dstb, idxb, rows_v, acc_sh, semg, sems):
  c = lax.axis_index("c")
  s = lax.axis_index("s")
  base_node = c * HALF

  zeros16 = jnp.zeros((16,), jnp.float32)
  def _zero(i, _):
    r = i // 4
    col = i % 4
    rows_v[0, r, pl.ds(col * 16, 16)] = zeros16
    return 0
  lax.fori_loop(0, 128 * 4, _zero, 0)
  def _zcopy(k, _):
    pltpu.sync_copy(rows_v.at[0],
                    acc_sh.at[pl.ds(s * STRIPE + k * 128, 128)])
    return 0
  lax.fori_loop(0, 13, _zcopy, 0)
  pltpu.sync_copy(rows_v.at[0].at[pl.ds(0, 1)],
                  acc_sh.at[pl.ds(s * STRIPE + 1664, 1)])
  plsc.subcore_barrier()

  base_row = s * (PT_E // 128)
  def _edge(b, _):
    row0 = base_row + b * 8
    pltpu.sync_copy(src_hbm.at[pl.ds(row0, 8)], srcb)
    gd = {0: pltpu.async_copy(ht_hbm.at[srcb.at[0]], rows_v.at[0], semg)}
    pltpu.sync_copy(dst_hbm.at[pl.ds(row0, 8)], dstb)
    def _idx(i, _):
      r = i // 8
      k = i % 8
      d = dstb[r, pl.ds(k * 16, 16)]
      loc = d - base_node
      ok = (loc >= 0) & (loc < HALF)
      idxb[r, pl.ds(k * 16, 16)] = jnp.where(ok, loc, HALF)
      return 0
    lax.fori_loop(0, 64, _idx, 0)
    sd = {}
    for j in range(8):
      gd[j].wait()
      if j >= 1:
        sd[j - 1].wait()
      if j < 7:
        gd[j + 1] = pltpu.async_copy(ht_hbm.at[srcb.at[j + 1]],
                                     rows_v.at[(j + 1) % 2], semg)
      sd[j] = pltpu.async_copy(rows_v.at[j % 2], acc_sh.at[idxb.at[j]],
                               sems, add=True)
    sd[7].wait()
    return 0
  lax.fori_loop(0, PT_E // 128 // 8, _edge, 0)
  plsc.subcore_barrier()

  out_base = c * HALF + s * (HALF // NS)
  sp_base = s * (HALF // NS)
  def _wb(k, _):
    pltpu.sync_copy(acc_sh.at[pl.ds(sp_base + k * 128, 128)], rows_v.at[0])
    pltpu.sync_copy(rows_v.at[0],
                    acc_out.at[pl.ds(out_base + k * 128, 128)])
    return 0
  lax.fori_loop(0, 13, _wb, 0)


_edges = pl.kernel(
    _sc_edges,
    out_type=jax.ShapeDtypeStruct((NPAD, D_EMB), jnp.float32),
    mesh=plsc.VectorSubcoreMesh(**_MESH),
    scratch_types=[
        pltpu.VMEM((8, 128), jnp.int32),
        pltpu.VMEM((8, 128), jnp.int32),
        pltpu.VMEM((8, 128), jnp.int32),
        pltpu.VMEM((2, 128, D_EMB), jnp.float32),
        pltpu.VMEM_SHARED((ACC_ROWS, D_EMB), jnp.float32),
        pltpu.SemaphoreType.DMA,
        pltpu.SemaphoreType.DMA,
    ],
    compiler_params=pltpu.CompilerParams(use_tc_tiling_on_sc=False),
)


def _tc_final(acc_ref, ht_ref, dinv_ref, wg_ref, bg_ref, wl_ref, bl_ref,
              out_ref, s_scr):
  g = pl.program_id(0)

  @pl.when(g == 0)
  def _():
    s_scr[...] = jnp.zeros_like(s_scr)

  agg = (acc_ref[...] + ht_ref[...]) * dinv_ref[...]
  aggh = jnp.dot(agg, wg_ref[...], preferred_element_type=jnp.float32)
  r = jnp.maximum(aggh + bg_ref[...], 0.0)
  rows = lax.broadcasted_iota(jnp.int32, (_BK, 1), 0) + g * _BK
  r = jnp.where(rows < N, r, 0.0)
  s_scr[...] += jnp.sum(r, axis=0, keepdims=True)

  @pl.when(g == _GB - 1)
  def _():
    out_ref[...] = (
        jnp.dot(s_scr[...], wl_ref[...], preferred_element_type=jnp.float32)
        + jnp.float32(N) * bl_ref[...])


_final = pl.pallas_call(
    _tc_final,
    grid=(_GB,),
    in_specs=[
        pl.BlockSpec((_BK, D_EMB), lambda g: (g, 0)),
        pl.BlockSpec((_BK, D_EMB), lambda g: (g, 0)),
        pl.BlockSpec((_BK, 1), lambda g: (g, 0)),
        pl.BlockSpec((D_EMB, D_HID), lambda g: (0, 0)),
        pl.BlockSpec((1, D_HID), lambda g: (0, 0)),
        pl.BlockSpec((D_HID, D_OUT), lambda g: (0, 0)),
        pl.BlockSpec((1, D_OUT), lambda g: (0, 0)),
    ],
    out_specs=pl.BlockSpec((1, D_OUT), lambda g: (0, 0)),
    out_shape=jax.ShapeDtypeStruct((1, D_OUT), jnp.float32),
    scratch_shapes=[pltpu.VMEM((1, D_HID), jnp.float32)],
)


@jax.jit
def kernel(x, edge_index, table, W_gcn, b_gcn, W_lin, b_lin):
  xp = jnp.concatenate([x[:, 0], jnp.zeros((NPAD - N,), jnp.int32)])
  src = jnp.concatenate([edge_index[0], jnp.zeros((EPAD - E,), jnp.int32)])
  dst = jnp.concatenate(
      [edge_index[1], jnp.full((EPAD - E,), -1, jnp.int32)])

  h, deg_part = _gather_deg(xp, table, dst)
  dinv, ht = _prep(deg_part.T, h)
  acc = _edges(ht, src.reshape(EPAD // 128, 128),
               dst.reshape(EPAD // 128, 128))
  out = _final(acc, ht, dinv, W_gcn, b_gcn.reshape(1, D_HID),
               W_lin, b_lin.reshape(1, D_OUT))
  return out.reshape(1, 1, D_OUT)


# speedup vs baseline: 19.3236x; 1.0113x over previous
import functools

import jax
import jax.numpy as jnp
from jax import lax
from jax.experimental import pallas as pl
from jax.experimental.pallas import tpu as pltpu
from jax.experimental.pallas import tpu_sc as plsc

N = 50000
E = 800000
VOCAB = 100000
D_EMB = 64
D_HID = 128
D_OUT = 64

NC = 2
NS = 16
NW = NC * NS

NPAD = 53248
PT_N = NPAD // NW
HALF = NPAD // NC
ACC_ROWS = HALF + 16
STRIPE = ACC_ROWS // NS

EPAD = 802816
PT_E_DEG = EPAD // NW
PT_E = EPAD // NS

_MESH = dict(core_axis_name="c", subcore_axis_name="s", num_cores=NC,
             num_subcores=NS)


DEG_ROWS = NPAD + 128
DEG_STRIPE = DEG_ROWS // NS


def _sc_gather_deg(xp_hbm, table_hbm, dst_hbm, h_out, deg_out,
                   idx_v, rows_v, dstb_v, idxb_v, ones_v, zb_v, deg_sh, sem):
  c = lax.axis_index("c")
  s = lax.axis_index("s")
  wid = s * NC + c

  zeros16 = jnp.zeros((16,), jnp.float32)
  ones16 = jnp.ones((16,), jnp.float32)
  def _zero(i, _):
    zb_v[pl.ds(i * 16, 16)] = zeros16
    return 0
  lax.fori_loop(0, 1024 // 16, _zero, 0)
  for j in range(8):
    ones_v[pl.ds(j * 16, 16)] = ones16
  for off, sz in ((0, 1024), (1024, 1024), (2048, 1024), (3072, 264)):
    pltpu.sync_copy(zb_v.at[pl.ds(0, sz)],
                    deg_sh.at[pl.ds(s * DEG_STRIPE + off, sz)])
  plsc.subcore_barrier()

  base_n = wid * PT_N
  for k in range(PT_N // 128):
    off = base_n + k * 128
    pltpu.sync_copy(xp_hbm.at[pl.ds(off, 128)], idx_v)
    pltpu.async_copy(table_hbm.at[idx_v], rows_v, sem).wait()
    pltpu.sync_copy(rows_v, h_out.at[pl.ds(off, 128)])

  base_e = (s * NC + c) * PT_E_DEG
  def _hist(k, _):
    pltpu.sync_copy(dst_hbm.at[pl.ds(base_e + k * 128, 128)], dstb_v)
    for j in range(8):
      d = dstb_v[pl.ds(j * 16, 16)]
      idxb_v[pl.ds(j * 16, 16)] = jnp.where(d >= 0, d, NPAD)
    pltpu.sync_copy(ones_v, deg_sh.at[idxb_v], add=True)
    return 0
  lax.fori_loop(0, PT_E_DEG // 128, _hist, 0)
  plsc.subcore_barrier()

  dbase = s * (NPAD // NS)
  for off in (0, 1024, 2048):
    sz = 1024 if off < 2048 else NPAD // NS - 2048
    pltpu.sync_copy(deg_sh.at[pl.ds(dbase + off, sz)], zb_v.at[pl.ds(0, sz)])
    pltpu.sync_copy(zb_v.at[pl.ds(0, sz)],
                    deg_out.at[c].at[pl.ds(dbase + off, sz)])


_gather_deg = pl.kernel(
    _sc_gather_deg,
    out_type=(jax.ShapeDtypeStruct((NPAD, D_EMB), jnp.float32),
              jax.ShapeDtypeStruct((NC, NPAD), jnp.float32)),
    mesh=plsc.VectorSubcoreMesh(**_MESH),
    scratch_types=[
        pltpu.VMEM((128,), jnp.int32),
        pltpu.VMEM((128, D_EMB), jnp.float32),
        pltpu.VMEM((128,), jnp.int32),
        pltpu.VMEM((128,), jnp.int32),
        pltpu.VMEM((128,), jnp.float32),
        pltpu.VMEM((1024,), jnp.float32),
        pltpu.VMEM_SHARED((DEG_ROWS,), jnp.float32),
        pltpu.SemaphoreType.DMA,
    ],
    compiler_params=pltpu.CompilerParams(use_tc_tiling_on_sc=False),
)


_BK = 4096
_GB = NPAD // _BK


def _tc_prep(degT_ref, h_ref, dinv_ref, ht_ref):
  deg = jnp.sum(degT_ref[...], axis=1, keepdims=True) + 1.0
  dinv = lax.rsqrt(deg)
  dinv_ref[...] = dinv
  ht_ref[...] = h_ref[...] * dinv


_prep = pl.pallas_call(
    _tc_prep,
    grid=(_GB,),
    in_specs=[
        pl.BlockSpec((_BK, NC), lambda g: (g, 0)),
        pl.BlockSpec((_BK, D_EMB), lambda g: (g, 0)),
    ],
    out_specs=[
        pl.BlockSpec((_BK, 1), lambda g: (g, 0)),
        pl.BlockSpec((_BK, D_EMB), lambda g: (g, 0)),
    ],
    out_shape=[
        jax.ShapeDtypeStruct((NPAD, 1), jnp.float32),
        jax.ShapeDtypeStruct((NPAD, D_EMB), jnp.float32),
    ],
)


def _sc_edges(ht_hbm, src_hbm, dst_hbm, acc_out,
              srcb, dstb, idxb, rows_v, acc_sh,
              semg0, semg1, sems0, sems1):
  semg = (semg0, semg1)
  sems = (sems0, sems1)
  c = lax.axis_index("c")
  s = lax.axis_index("s")
  base_node = c * HALF

  zeros16 = jnp.zeros((16,), jnp.float32)
  def _zero(i, _):
    r = i // 4
    col = i % 4
    rows_v[0, r, pl.ds(col * 16, 16)] = zeros16
    return 0
  lax.fori_loop(0, 128 * 4, _zero, 0)
  def _zcopy(k, _):
    pltpu.sync_copy(rows_v.at[0],
                    acc_sh.at[pl.ds(s * STRIPE + k * 128, 128)])
    return 0
  lax.fori_loop(0, 13, _zcopy, 0)
  pltpu.sync_copy(rows_v.at[0].at[pl.ds(0, 1)],
                  acc_sh.at[pl.ds(s * STRIPE + 1664, 1)])
  plsc.subcore_barrier()

  BLK = 14
  base_row = s * (PT_E // 128)
  def _edge(b, _):
    row0 = base_row + b * BLK
    pltpu.sync_copy(src_hbm.at[pl.ds(row0, BLK)], srcb)
    gd = {0: pltpu.async_copy(ht_hbm.at[srcb.at[0]], rows_v.at[0], semg[0]),
          1: pltpu.async_copy(ht_hbm.at[srcb.at[1]], rows_v.at[1], semg[1])}
    pltpu.sync_copy(dst_hbm.at[pl.ds(row0, BLK)], dstb)
    def _idx(i, _):
      r = i // 8
      k = i % 8
      d = dstb[r, pl.ds(k * 16, 16)]
      loc = d - base_node
      ok = (loc >= 0) & (loc < HALF)
      idxb[r, pl.ds(k * 16, 16)] = jnp.where(ok, loc, HALF)
      return 0
    lax.fori_loop(0, BLK * 8, _idx, 0)
    sd = {}
    for j in range(BLK):
      if j >= 1:
        sd[j - 1].wait()
      if j + 2 < BLK:
        gd[j + 2] = pltpu.async_copy(ht_hbm.at[srcb.at[j + 2]],
                                     rows_v.at[(j + 2) % 2], semg[j % 2])
      gd[j].wait()
      sd[j] = pltpu.async_copy(rows_v.at[j % 2], acc_sh.at[idxb.at[j]],
                               sems[j % 2], add=True)
    sd[BLK - 1].wait()
    return 0
  lax.fori_loop(0, PT_E // 128 // BLK, _edge, 0)
  plsc.subcore_barrier()

  out_base = c * HALF + s * (HALF // NS)
  sp_base = s * (HALF // NS)
  def _wb(k, _):
    pltpu.sync_copy(acc_sh.at[pl.ds(sp_base + k * 128, 128)], rows_v.at[0])
    pltpu.sync_copy(rows_v.at[0],
                    acc_out.at[pl.ds(out_base + k * 128, 128)])
    return 0
  lax.fori_loop(0, 13, _wb, 0)


_edges = pl.kernel(
    _sc_edges,
    out_type=jax.ShapeDtypeStruct((NPAD, D_EMB), jnp.float32),
    mesh=plsc.VectorSubcoreMesh(**_MESH),
    scratch_types=[
        pltpu.VMEM((14, 128), jnp.int32),
        pltpu.VMEM((14, 128), jnp.int32),
        pltpu.VMEM((14, 128), jnp.int32),
        pltpu.VMEM((2, 128, D_EMB), jnp.float32),
        pltpu.VMEM_SHARED((ACC_ROWS, D_EMB), jnp.float32),
        pltpu.SemaphoreType.DMA,
        pltpu.SemaphoreType.DMA,
        pltpu.SemaphoreType.DMA,
        pltpu.SemaphoreType.DMA,
    ],
    compiler_params=pltpu.CompilerParams(use_tc_tiling_on_sc=False),
)


def _tc_final(acc_ref, ht_ref, dinv_ref, wg_ref, bg_ref, wl_ref, bl_ref,
              out_ref, s_scr):
  g = pl.program_id(0)

  @pl.when(g == 0)
  def _():
    s_scr[...] = jnp.zeros_like(s_scr)

  agg = (acc_ref[...] + ht_ref[...]) * dinv_ref[...]
  aggh = jnp.dot(agg, wg_ref[...], preferred_element_type=jnp.float32)
  r = jnp.maximum(aggh + bg_ref[...], 0.0)
  rows = lax.broadcasted_iota(jnp.int32, (_BK, 1), 0) + g * _BK
  r = jnp.where(rows < N, r, 0.0)
  s_scr[...] += jnp.sum(r, axis=0, keepdims=True)

  @pl.when(g == _GB - 1)
  def _():
    out_ref[...] = (
        jnp.dot(s_scr[...], wl_ref[...], preferred_element_type=jnp.float32)
        + jnp.float32(N) * bl_ref[...])


_final = pl.pallas_call(
    _tc_final,
    grid=(_GB,),
    in_specs=[
        pl.BlockSpec((_BK, D_EMB), lambda g: (g, 0)),
        pl.BlockSpec((_BK, D_EMB), lambda g: (g, 0)),
        pl.BlockSpec((_BK, 1), lambda g: (g, 0)),
        pl.BlockSpec((D_EMB, D_HID), lambda g: (0, 0)),
        pl.BlockSpec((1, D_HID), lambda g: (0, 0)),
        pl.BlockSpec((D_HID, D_OUT), lambda g: (0, 0)),
        pl.BlockSpec((1, D_OUT), lambda g: (0, 0)),
    ],
    out_specs=pl.BlockSpec((1, D_OUT), lambda g: (0, 0)),
    out_shape=jax.ShapeDtypeStruct((1, D_OUT), jnp.float32),
    scratch_shapes=[pltpu.VMEM((1, D_HID), jnp.float32)],
)


@jax.jit
def kernel(x, edge_index, table, W_gcn, b_gcn, W_lin, b_lin):
  xp = jnp.concatenate([x[:, 0], jnp.zeros((NPAD - N,), jnp.int32)])
  src = jnp.concatenate([edge_index[0], jnp.zeros((EPAD - E,), jnp.int32)])
  dst = jnp.concatenate(
      [edge_index[1], jnp.full((EPAD - E,), -1, jnp.int32)])

  h, deg_part = _gather_deg(xp, table, dst)
  dinv, ht = _prep(deg_part.T, h)
  acc = _edges(ht, src.reshape(EPAD // 128, 128),
               dst.reshape(EPAD // 128, 128))
  out = _final(acc, ht, dinv, W_gcn, b_gcn.reshape(1, D_HID),
               W_lin, b_lin.reshape(1, D_OUT))
  return out.reshape(1, 1, D_OUT)


# speedup vs baseline: 21.2178x; 1.0980x over previous
import functools

import jax
import jax.numpy as jnp
from jax import lax
from jax.experimental import pallas as pl
from jax.experimental.pallas import tpu as pltpu
from jax.experimental.pallas import tpu_sc as plsc

N = 50000
E = 800000
VOCAB = 100000
D_EMB = 64
D_HID = 128
D_OUT = 64

NC = 2
NS = 16
NW = NC * NS

NPAD = 53248
PT_N = NPAD // NW
HALF = NPAD // NC
ACC_ROWS = HALF + 16
STRIPE = ACC_ROWS // NS

EPAD = 802816
PT_E_DEG = EPAD // NW
PT_E = EPAD // NS

_MESH = dict(core_axis_name="c", subcore_axis_name="s", num_cores=NC,
             num_subcores=NS)


DEG_ROWS = NPAD + 128
DEG_STRIPE = DEG_ROWS // NS


def _sc_gather_deg(xp_hbm, table_hbm, dst_hbm, h_out, deg_out,
                   xidxb, rows_v, dstrows, idxrows, ones_v, zb_v, deg_sh,
                   semg0, semg1, semw0, semw1, sems):
  semg = (semg0, semg1)
  semw = (semw0, semw1)
  c = lax.axis_index("c")
  s = lax.axis_index("s")
  wid = s * NC + c

  zeros16 = jnp.zeros((16,), jnp.float32)
  ones16 = jnp.ones((16,), jnp.float32)
  def _zero(i, _):
    zb_v[pl.ds(i * 16, 16)] = zeros16
    return 0
  lax.fori_loop(0, 1024 // 16, _zero, 0)
  for j in range(8):
    ones_v[pl.ds(j * 16, 16)] = ones16
  for off, sz in ((0, 1024), (1024, 1024), (2048, 1024), (3072, 264)):
    pltpu.sync_copy(zb_v.at[pl.ds(0, sz)],
                    deg_sh.at[pl.ds(s * DEG_STRIPE + off, sz)])
  plsc.subcore_barrier()

  base_n = wid * PT_N
  NCH = PT_N // 128
  pltpu.sync_copy(xp_hbm.at[pl.ds(wid * NCH, NCH)], xidxb)
  gd = {0: pltpu.async_copy(table_hbm.at[xidxb.at[0]], rows_v.at[0], semg[0]),
        1: pltpu.async_copy(table_hbm.at[xidxb.at[1]], rows_v.at[1], semg[1])}
  wd = {}
  for j in range(NCH):
    if j >= 1:
      wd[j - 1].wait()
    if j + 2 < NCH:
      gd[j + 2] = pltpu.async_copy(table_hbm.at[xidxb.at[j + 2]],
                                   rows_v.at[(j + 2) % 2], semg[j % 2])
    gd[j].wait()
    wd[j] = pltpu.async_copy(rows_v.at[j % 2],
                             h_out.at[pl.ds(base_n + j * 128, 128)],
                             semw[j % 2])
  wd[NCH - 1].wait()

  NR = PT_E_DEG // 128
  pltpu.sync_copy(dst_hbm.at[pl.ds(wid * NR, NR)], dstrows)
  def _idx(i, _):
    r = i // 8
    k = i % 8
    d = dstrows[r, pl.ds(k * 16, 16)]
    idxrows[r, pl.ds(k * 16, 16)] = jnp.where(d >= 0, d, NPAD)
    return 0
  lax.fori_loop(0, NR * 8, _idx, 0)
  def _fire(b, _):
    ds_ = [pltpu.async_copy(ones_v, deg_sh.at[idxrows.at[b * 14 + j]],
                            sems, add=True) for j in range(14)]
    for dsc in ds_:
      dsc.wait()
    return 0
  lax.fori_loop(0, NR // 14, _fire, 0)
  plsc.subcore_barrier()

  dbase = s * (NPAD // NS)
  for off in (0, 1024, 2048):
    sz = 1024 if off < 2048 else NPAD // NS - 2048
    pltpu.sync_copy(deg_sh.at[pl.ds(dbase + off, sz)], zb_v.at[pl.ds(0, sz)])
    pltpu.sync_copy(zb_v.at[pl.ds(0, sz)],
                    deg_out.at[c].at[pl.ds(dbase + off, sz)])


_gather_deg = pl.kernel(
    _sc_gather_deg,
    out_type=(jax.ShapeDtypeStruct((NPAD, D_EMB), jnp.float32),
              jax.ShapeDtypeStruct((NC, NPAD), jnp.float32)),
    mesh=plsc.VectorSubcoreMesh(**_MESH),
    scratch_types=[
        pltpu.VMEM((13, 128), jnp.int32),
        pltpu.VMEM((2, 128, D_EMB), jnp.float32),
        pltpu.VMEM((196, 128), jnp.int32),
        pltpu.VMEM((196, 128), jnp.int32),
        pltpu.VMEM((128,), jnp.float32),
        pltpu.VMEM((1024,), jnp.float32),
        pltpu.VMEM_SHARED((DEG_ROWS,), jnp.float32),
        pltpu.SemaphoreType.DMA,
        pltpu.SemaphoreType.DMA,
        pltpu.SemaphoreType.DMA,
        pltpu.SemaphoreType.DMA,
        pltpu.SemaphoreType.DMA,
    ],
    compiler_params=pltpu.CompilerParams(use_tc_tiling_on_sc=False),
)


_BK = 4096
_GB = NPAD // _BK


def _tc_prep(degT_ref, h_ref, dinv_ref, ht_ref):
  deg = jnp.sum(degT_ref[...], axis=1, keepdims=True) + 1.0
  dinv = lax.rsqrt(deg)
  dinv_ref[...] = dinv
  ht_ref[...] = h_ref[...] * dinv


_prep = pl.pallas_call(
    _tc_prep,
    grid=(_GB,),
    in_specs=[
        pl.BlockSpec((_BK, NC), lambda g: (g, 0)),
        pl.BlockSpec((_BK, D_EMB), lambda g: (g, 0)),
    ],
    out_specs=[
        pl.BlockSpec((_BK, 1), lambda g: (g, 0)),
        pl.BlockSpec((_BK, D_EMB), lambda g: (g, 0)),
    ],
    out_shape=[
        jax.ShapeDtypeStruct((NPAD, 1), jnp.float32),
        jax.ShapeDtypeStruct((NPAD, D_EMB), jnp.float32),
    ],
)


def _sc_edges(ht_hbm, src_hbm, dst_hbm, acc_out,
              srcb, dstb, idxb, rows_v, acc_sh,
              semg0, semg1, sems0, sems1):
  semg = (semg0, semg1)
  sems = (sems0, sems1)
  c = lax.axis_index("c")
  s = lax.axis_index("s")
  base_node = c * HALF

  zeros16 = jnp.zeros((16,), jnp.float32)
  def _zero(i, _):
    r = i // 4
    col = i % 4
    rows_v[0, r, pl.ds(col * 16, 16)] = zeros16
    return 0
  lax.fori_loop(0, 128 * 4, _zero, 0)
  def _zcopy(k, _):
    pltpu.sync_copy(rows_v.at[0],
                    acc_sh.at[pl.ds(s * STRIPE + k * 128, 128)])
    return 0
  lax.fori_loop(0, 13, _zcopy, 0)
  pltpu.sync_copy(rows_v.at[0].at[pl.ds(0, 1)],
                  acc_sh.at[pl.ds(s * STRIPE + 1664, 1)])
  plsc.subcore_barrier()

  BLK = 14
  base_row = s * (PT_E // 128)
  def _edge(b, _):
    row0 = base_row + b * BLK
    pltpu.sync_copy(src_hbm.at[pl.ds(row0, BLK)], srcb)
    gd = {0: pltpu.async_copy(ht_hbm.at[srcb.at[0]], rows_v.at[0], semg[0]),
          1: pltpu.async_copy(ht_hbm.at[srcb.at[1]], rows_v.at[1], semg[1])}
    pltpu.sync_copy(dst_hbm.at[pl.ds(row0, BLK)], dstb)
    def _idx(i, _):
      r = i // 8
      k = i % 8
      d = dstb[r, pl.ds(k * 16, 16)]
      loc = d - base_node
      ok = (loc >= 0) & (loc < HALF)
      idxb[r, pl.ds(k * 16, 16)] = jnp.where(ok, loc, HALF)
      return 0
    lax.fori_loop(0, BLK * 8, _idx, 0)
    sd = {}
    for j in range(BLK):
      if j >= 1:
        sd[j - 1].wait()
      if j + 2 < BLK:
        gd[j + 2] = pltpu.async_copy(ht_hbm.at[srcb.at[j + 2]],
                                     rows_v.at[(j + 2) % 2], semg[j % 2])
      gd[j].wait()
      sd[j] = pltpu.async_copy(rows_v.at[j % 2], acc_sh.at[idxb.at[j]],
                               sems[j % 2], add=True)
    sd[BLK - 1].wait()
    return 0
  lax.fori_loop(0, PT_E // 128 // BLK, _edge, 0)
  plsc.subcore_barrier()

  out_base = c * HALF + s * (HALF // NS)
  sp_base = s * (HALF // NS)
  def _wb(k, _):
    pltpu.sync_copy(acc_sh.at[pl.ds(sp_base + k * 128, 128)], rows_v.at[0])
    pltpu.sync_copy(rows_v.at[0],
                    acc_out.at[pl.ds(out_base + k * 128, 128)])
    return 0
  lax.fori_loop(0, 13, _wb, 0)


_edges = pl.kernel(
    _sc_edges,
    out_type=jax.ShapeDtypeStruct((NPAD, D_EMB), jnp.float32),
    mesh=plsc.VectorSubcoreMesh(**_MESH),
    scratch_types=[
        pltpu.VMEM((14, 128), jnp.int32),
        pltpu.VMEM((14, 128), jnp.int32),
        pltpu.VMEM((14, 128), jnp.int32),
        pltpu.VMEM((2, 128, D_EMB), jnp.float32),
        pltpu.VMEM_SHARED((ACC_ROWS, D_EMB), jnp.float32),
        pltpu.SemaphoreType.DMA,
        pltpu.SemaphoreType.DMA,
        pltpu.SemaphoreType.DMA,
        pltpu.SemaphoreType.DMA,
    ],
    compiler_params=pltpu.CompilerParams(use_tc_tiling_on_sc=False),
)


def _tc_final(acc_ref, ht_ref, dinv_ref, wg_ref, bg_ref, wl_ref, bl_ref,
              out_ref, s_scr):
  g = pl.program_id(0)

  @pl.when(g == 0)
  def _():
    s_scr[...] = jnp.zeros_like(s_scr)

  agg = (acc_ref[...] + ht_ref[...]) * dinv_ref[...]
  aggh = jnp.dot(agg, wg_ref[...], preferred_element_type=jnp.float32)
  r = jnp.maximum(aggh + bg_ref[...], 0.0)
  rows = lax.broadcasted_iota(jnp.int32, (_BK, 1), 0) + g * _BK
  r = jnp.where(rows < N, r, 0.0)
  s_scr[...] += jnp.sum(r, axis=0, keepdims=True)

  @pl.when(g == _GB - 1)
  def _():
    out_ref[...] = (
        jnp.dot(s_scr[...], wl_ref[...], preferred_element_type=jnp.float32)
        + jnp.float32(N) * bl_ref[...])


_final = pl.pallas_call(
    _tc_final,
    grid=(_GB,),
    in_specs=[
        pl.BlockSpec((_BK, D_EMB), lambda g: (g, 0)),
        pl.BlockSpec((_BK, D_EMB), lambda g: (g, 0)),
        pl.BlockSpec((_BK, 1), lambda g: (g, 0)),
        pl.BlockSpec((D_EMB, D_HID), lambda g: (0, 0)),
        pl.BlockSpec((1, D_HID), lambda g: (0, 0)),
        pl.BlockSpec((D_HID, D_OUT), lambda g: (0, 0)),
        pl.BlockSpec((1, D_OUT), lambda g: (0, 0)),
    ],
    out_specs=pl.BlockSpec((1, D_OUT), lambda g: (0, 0)),
    out_shape=jax.ShapeDtypeStruct((1, D_OUT), jnp.float32),
    scratch_shapes=[pltpu.VMEM((1, D_HID), jnp.float32)],
)


@jax.jit
def kernel(x, edge_index, table, W_gcn, b_gcn, W_lin, b_lin):
  xp = jnp.concatenate([x[:, 0], jnp.zeros((NPAD - N,), jnp.int32)])
  src = jnp.concatenate([edge_index[0], jnp.zeros((EPAD - E,), jnp.int32)])
  dst = jnp.concatenate(
      [edge_index[1], jnp.full((EPAD - E,), -1, jnp.int32)])

  src2 = src.reshape(EPAD // 128, 128)
  dst2 = dst.reshape(EPAD // 128, 128)
  h, deg_part = _gather_deg(xp.reshape(NPAD // 128, 128), table, dst2)
  dinv, ht = _prep(deg_part.T, h)
  acc = _edges(ht, src2, dst2)
  out = _final(acc, ht, dinv, W_gcn, b_gcn.reshape(1, D_HID),
               W_lin, b_lin.reshape(1, D_OUT))
  return out.reshape(1, 1, D_OUT)
